# Initial kernel scaffold; baseline (speedup 1.0000x reference)
import jax
import jax.numpy as jnp
from jax.experimental import pallas as pl


def kernel(atom_feats, pimg, params, b1_src, b1_dst, b2_src, b2_dst, g1_src, g1_dst, i2_src, i2_dst):
    raise NotImplementedError("write your pallas kernel here")


# speedup vs baseline: 40.4699x; 40.4699x over previous
import functools

import jax
import jax.numpy as jnp
from jax import lax
from jax.experimental import pallas as pl
from jax.experimental.pallas import tpu as pltpu
from jax.experimental.pallas import tpu_sc as plsc

N_A = 10000
N_C2 = 2000
E_BT = 640000
E_I2 = 32000
HID = 128
NH = 4
HD = 32
PI = 50
NS = 16
NC = 2

f32 = jnp.float32
i32 = jnp.int32

_MESH = dict(core_axis_name="c", subcore_axis_name="s",
             num_cores=NC, num_subcores=NS)


def _full16(v):
    return jnp.full((16,), v, i32)


_CHB = 64
_NCHB = E_BT // NS // _CHB


def _make_sc_gat_a():
    @functools.partial(
        pl.kernel,
        out_type=(jax.ShapeDtypeStruct((2 * E_BT * NH,), f32),
                  jax.ShapeDtypeStruct((NC * NS * NH * N_A,), f32)),
        mesh=plsc.VectorSubcoreMesh(**_MESH),
        compiler_params=pltpu.CompilerParams(needs_layout_passes=False),
        scratch_types=[
            pltpu.VMEM((N_A * 8,), f32),
            pltpu.VMEM((NH * N_A,), f32),
            pltpu.VMEM((_CHB * NH,), f32),
            pltpu.VMEM((_CHB,), i32),
            pltpu.VMEM((_CHB,), i32),
        ])
    def k(eler, srcs, dsts, zdn, ee_out, dnm,
          eler_t, denp, eeflat, sbuf, dbuf):
        c = lax.axis_index("c")
        s = lax.axis_index("s")
        pltpu.sync_copy(zdn, denp)
        pltpu.sync_copy(eler.at[pl.ds(pl.multiple_of(c * (N_A * 8), 8),
                                      N_A * 8)], eler_t)
        base = c * E_BT + s * (E_BT // NS)
        iota = lax.iota(i32, 16)

        def chunk(kk, carry):
            off = base + kk * _CHB
            pltpu.sync_copy(srcs.at[pl.ds(off, _CHB)], sbuf)
            pltpu.sync_copy(dsts.at[pl.ds(off, _CHB)], dbuf)
            for j in range(_CHB // 16):
                sv8 = sbuf[pl.ds(j * 16, 16)] * 8
                dv = dbuf[pl.ds(j * 16, 16)]
                dv8 = dv * 8
                rr = iota + j * 16
                for h in range(NH):
                    el = plsc.load_gather(eler_t, [sv8 + h])
                    er = plsc.load_gather(eler_t, [dv8 + (h + NH)])
                    e = el + er
                    e = jnp.where(e >= 0.0, e, 0.2 * e)
                    ee = jnp.exp(e)
                    plsc.addupdate_scatter(denp, [dv + h * N_A], ee)
                    plsc.store_scatter(eeflat, [rr * NH + h], ee)
            pltpu.sync_copy(eeflat,
                            ee_out.at[pl.ds(off * NH, _CHB * NH)])
            return carry

        lax.fori_loop(0, _NCHB, chunk, 0)
        pltpu.sync_copy(
            denp,
            dnm.at[pl.ds(pl.multiple_of((c * NS + s) * (NH * N_A), 8),
                         NH * N_A)])

    return k


def _make_sc_gat_b():
    @functools.partial(
        pl.kernel,
        out_type=jax.ShapeDtypeStruct((2 * N_A, HID), f32),
        mesh=plsc.VectorSubcoreMesh(**_MESH),
        compiler_params=pltpu.CompilerParams(needs_layout_passes=False),
        scratch_types=[
            pltpu.VMEM((_CHB, HID), f32),
            pltpu.VMEM((_CHB * NH,), f32),
            pltpu.VMEM((_CHB,), i32),
            pltpu.VMEM((_CHB,), i32),
            pltpu.VMEM((_CHB,), i32),
            pltpu.VMEM_SHARED((N_A, HID), f32),
            pltpu.SemaphoreType.DMA,
        ])
    def k(feats, eehbm, srcs, dsts, z128, rstu,
          fbuf, eeflat, sbuf, dbuf, gbuf, acc, sem):
        c = lax.axis_index("c")
        s = lax.axis_index("s")
        r0 = pl.multiple_of(s * 640, 8)
        g0 = pl.multiple_of(c * N_A + s * 640, 8)

        @pl.when(s < NS - 1)
        def _():
            pltpu.sync_copy(z128, acc.at[pl.ds(r0, 640)])

        @pl.when(s == NS - 1)
        def _():
            pltpu.sync_copy(z128.at[pl.ds(0, 400)], acc.at[pl.ds(r0, 400)])

        plsc.subcore_barrier()
        base = c * E_BT + s * (E_BT // NS)
        cbase = c * N_A

        def chunk(kk, carry):
            off = base + kk * _CHB
            pltpu.sync_copy(srcs.at[pl.ds(off, _CHB)], sbuf)
            pltpu.sync_copy(dsts.at[pl.ds(off, _CHB)], dbuf)
            for j in range(_CHB // 16):
                gbuf[pl.ds(j * 16, 16)] = sbuf[pl.ds(j * 16, 16)] + cbase
            cp = pltpu.async_copy(feats.at[gbuf], fbuf, sem)
            pltpu.sync_copy(eehbm.at[pl.ds(off * NH, _CHB * NH)], eeflat)
            cp.wait()
            for ei in range(_CHB):
                aa = [plsc.load_gather(eeflat, [_full16(ei * NH + h)])
                      for h in range(NH)]
                for v in range(HID // 16):
                    fbuf[ei, pl.ds(v * 16, 16)] = (
                        fbuf[ei, pl.ds(v * 16, 16)] * aa[v * 16 // HD])
            pltpu.sync_copy(fbuf, acc.at[dbuf], add=True)
            return carry

        lax.fori_loop(0, _NCHB, chunk, 0)
        plsc.subcore_barrier()

        @pl.when(s < NS - 1)
        def _():
            pltpu.sync_copy(acc.at[pl.ds(r0, 640)], rstu.at[pl.ds(g0, 640)])

        @pl.when(s == NS - 1)
        def _():
            pltpu.sync_copy(acc.at[pl.ds(r0, 400)], rstu.at[pl.ds(g0, 400)])

    return k


def _make_sc_gin_bt():
    @functools.partial(
        pl.kernel,
        out_type=jax.ShapeDtypeStruct((2 * N_A, HID), f32),
        mesh=plsc.VectorSubcoreMesh(**_MESH),
        compiler_params=pltpu.CompilerParams(needs_layout_passes=False),
        scratch_types=[
            pltpu.VMEM((_CHB, HID), f32),
            pltpu.VMEM((_CHB,), i32),
            pltpu.VMEM((_CHB,), i32),
            pltpu.VMEM((_CHB,), i32),
            pltpu.VMEM_SHARED((N_A, HID), f32),
            pltpu.SemaphoreType.DMA,
        ])
    def k(table, srcs, dsts, z128, agg,
          fbuf, sbuf, dbuf, gbuf, acc, sem):
        c = lax.axis_index("c")
        s = lax.axis_index("s")
        r0 = pl.multiple_of(s * 640, 8)
        g0 = pl.multiple_of(c * N_A + s * 640, 8)

        @pl.when(s < NS - 1)
        def _():
            pltpu.sync_copy(z128, acc.at[pl.ds(r0, 640)])

        @pl.when(s == NS - 1)
        def _():
            pltpu.sync_copy(z128.at[pl.ds(0, 400)], acc.at[pl.ds(r0, 400)])

        plsc.subcore_barrier()
        base = c * E_BT + s * (E_BT // NS)
        cbase = c * N_A

        def chunk(kk, carry):
            off = base + kk * _CHB
            pltpu.sync_copy(srcs.at[pl.ds(off, _CHB)], sbuf)
            pltpu.sync_copy(dsts.at[pl.ds(off, _CHB)], dbuf)
            for j in range(_CHB // 16):
                gbuf[pl.ds(j * 16, 16)] = sbuf[pl.ds(j * 16, 16)] + cbase
            pltpu.async_copy(table.at[gbuf], fbuf, sem).wait()
            pltpu.sync_copy(fbuf, acc.at[dbuf], add=True)
            return carry

        lax.fori_loop(0, _NCHB, chunk, 0)
        plsc.subcore_barrier()

        @pl.when(s < NS - 1)
        def _():
            pltpu.sync_copy(acc.at[pl.ds(r0, 640)], agg.at[pl.ds(g0, 640)])

        @pl.when(s == NS - 1)
        def _():
            pltpu.sync_copy(acc.at[pl.ds(r0, 400)], agg.at[pl.ds(g0, 400)])

    return k


def _make_sc_g1max():
    @functools.partial(
        pl.kernel,
        out_type=jax.ShapeDtypeStruct((16 * N_C2 * 16,), f32),
        mesh=plsc.VectorSubcoreMesh(**_MESH),
        compiler_params=pltpu.CompilerParams(needs_layout_passes=False),
        scratch_types=[
            pltpu.VMEM((400, HID), f32),
            pltpu.VMEM((N_C2 * 16,), f32),
            pltpu.VMEM((400,), i32),
        ])
    def k(hA, g1dst, zmax, out, buf, acc, dstb):
        c = lax.axis_index("c")
        s = lax.axis_index("s")
        iota = lax.iota(i32, 16)

        @pl.when(s < 8)
        def _():
            pltpu.sync_copy(zmax, acc)
            cols = s * 16 + iota

            def rchunk(k2, carry):
                pltpu.sync_copy(
                    hA.at[pl.ds(pl.multiple_of(c * N_A + k2 * 400, 8), 400)],
                    buf)
                pltpu.sync_copy(g1dst.at[pl.ds(k2 * 400, 400)], dstb)

                def grp(g, carry2):
                    dv = dstb[pl.ds(g * 16, 16)]
                    nb = g * 16
                    for t in range(16):
                        di = _full16(dv[t]) * 16 + iota
                        v = plsc.load_gather(buf, [_full16(nb + t), cols])
                        cur = plsc.load_gather(acc, [di])
                        plsc.store_scatter(acc, [di], jnp.maximum(cur, v))
                    return carry2

                lax.fori_loop(0, 400 // 16, grp, 0)
                return carry

            lax.fori_loop(0, N_A // 400, rchunk, 0)
            pltpu.sync_copy(
                acc,
                out.at[pl.ds(pl.multiple_of((c * 8 + s) * (N_C2 * 16), 8),
                             N_C2 * 16)])

    return k


N_P = 2048
E_I2P = 32768


def _make_sc_i2(D):
    P = D // HID
    ept = E_I2P // (NC * NS)
    ch = 64

    @functools.partial(
        pl.kernel,
        out_type=jax.ShapeDtypeStruct((2 * P * N_P, HID), f32),
        mesh=plsc.VectorSubcoreMesh(**_MESH),
        compiler_params=pltpu.CompilerParams(needs_layout_passes=False),
        scratch_types=(
            [pltpu.VMEM((ch * P, HID), f32),
             pltpu.VMEM((ch,), i32),
             pltpu.VMEM((ch,), i32)]
            + [pltpu.VMEM((ch,), i32) for _ in range(2 * P)]
            + [pltpu.VMEM_SHARED((P * N_P, HID), f32),
               pltpu.SemaphoreType.DMA]),
    )
    def k(table, srcs, dsts, zrows, out, fbuf, sbuf, dbuf, *rest):
        idxs = rest[:2 * P]
        acc, sem = rest[2 * P], rest[2 * P + 1]
        c = lax.axis_index("c")
        s = lax.axis_index("s")
        rows = P * N_P // NS
        r0 = pl.multiple_of(s * rows, 8)
        g0 = pl.multiple_of(c * (P * N_P) + s * rows, 8)
        pltpu.sync_copy(zrows, acc.at[pl.ds(r0, rows)])
        plsc.subcore_barrier()
        base = (c * NS + s) * ept

        def chunk(kk, carry):
            off = base + kk * ch
            pltpu.sync_copy(srcs.at[pl.ds(off, ch)], sbuf)
            pltpu.sync_copy(dsts.at[pl.ds(off, ch)], dbuf)
            for p in range(P):
                for j in range(ch // 16):
                    sl = pl.ds(j * 16, 16)
                    idxs[p][sl] = sbuf[sl] * P + p
                    idxs[P + p][sl] = dbuf[sl] * P + p
            cps = [pltpu.async_copy(table.at[idxs[p]],
                                    fbuf.at[pl.ds(p * ch, ch)], sem)
                   for p in range(P)]
            for cp in cps:
                cp.wait()
            for p in range(P):
                pltpu.sync_copy(fbuf.at[pl.ds(p * ch, ch)],
                                acc.at[idxs[P + p]], add=True)
            return carry

        lax.fori_loop(0, ept // ch, chunk, 0)
        plsc.subcore_barrier()
        pltpu.sync_copy(acc.at[pl.ds(r0, rows)], out.at[pl.ds(g0, rows)])

    return k


@functools.lru_cache(maxsize=None)
def _sc_kernels():
    return (_make_sc_gat_a(), _make_sc_gat_b(), _make_sc_gin_bt(),
            _make_sc_g1max(), _make_sc_i2(384), _make_sc_i2(128))


def _tc(body, out_shape):
    return pl.pallas_call(body, out_shape=out_shape)


def _tca(x_ref, w_ref, m_ref, feats_ref, eler_ref):
    x = x_ref[...]
    for r in range(2):
        f = jnp.dot(x, w_ref[r], preferred_element_type=f32)
        feats_ref[pl.ds(r * N_A, N_A), :] = f
        eler_ref[pl.ds(r * N_A, N_A), :] = jnp.dot(
            f, m_ref[r], preferred_element_type=f32)


def _tcd(rstu_ref, d1_ref, d2_ref, e4_ref, bias_ref, out_ref):
    for r, dref in ((0, d1_ref), (1, d2_ref)):
        dn = jnp.sum(dref[...], axis=0)
        den = jnp.maximum(
            jax.lax.dot_general(dn, e4_ref[...], (((0,), (0,)), ((), ())),
                                preferred_element_type=f32), 1e-9)
        x = rstu_ref[pl.ds(r * N_A, N_A), :] / den
        out_ref[pl.ds(r * N_A, N_A), :] = jnp.maximum(
            x + bias_ref[pl.ds(r, 1), :], 0.0)


def _bn_mlp(h, w1, b1, g, be, w2, b2):
    h1 = jnp.dot(h, w1, preferred_element_type=f32) + b1
    mu = jnp.mean(h1, axis=0, keepdims=True)
    var = jnp.mean((h1 - mu) ** 2, axis=0, keepdims=True)
    hn = (h1 - mu) / jnp.sqrt(var + 1e-5) * g + be
    h2 = jnp.maximum(hn, 0.0)
    return jnp.dot(h2, w2, preferred_element_type=f32) + b2


def _tcf(hgat_ref, agg_ref, eps_ref, w1_ref, b1_ref, g_ref, be_ref,
         w2_ref, b2_ref, h_ref, m_ref):
    for r in range(2):
        sl = pl.ds(r * N_A, N_A)
        h = ((1.0 + eps_ref[pl.ds(r, 1), :]) * hgat_ref[sl, :]
             + agg_ref[sl, :])
        hr = jnp.maximum(
            _bn_mlp(h, w1_ref[r], b1_ref[pl.ds(r, 1), :], g_ref[pl.ds(r, 1), :],
                    be_ref[pl.ds(r, 1), :], w2_ref[r], b2_ref[pl.ds(r, 1), :]),
            0.0)
        h_ref[sl, :] = hr
        m_ref[pl.ds(r, 1), :] = jnp.mean(hr, axis=0, keepdims=True)


def _tch(hcb_ref, pimg_ref, out_ref):
    for i in range(16):
        off = 128 * (i // 8) + 16 * (i % 8)
        out_ref[pl.ds(0, N_C2), pl.ds(off, 16)] = hcb_ref[
            pl.ds(i * N_C2, N_C2), :]
    out_ref[pl.ds(0, N_C2), pl.ds(256, PI)] = pimg_ref[...]
    out_ref[pl.ds(0, N_C2), pl.ds(256 + PI, 384 - 256 - PI)] = jnp.zeros(
        (N_C2, 384 - 256 - PI), f32)
    out_ref[pl.ds(N_C2, N_P - N_C2), :] = jnp.zeros((N_P - N_C2, 384), f32)


def _tcj(t_ref, agg_ref, eps_ref, w1_ref, b1_ref, g_ref, be_ref,
         w2_ref, b2_ref, out_ref):
    a = agg_ref[pl.ds(0, N_C2), :] + agg_ref[pl.ds(N_P, N_C2), :]
    h = (1.0 + eps_ref[...]) * t_ref[pl.ds(0, N_C2), :] + a
    out_ref[pl.ds(0, N_C2), :] = jnp.maximum(
        _bn_mlp(h, w1_ref[...], b1_ref[...], g_ref[...], be_ref[...],
                w2_ref[...], b2_ref[...]), 0.0)
    out_ref[pl.ds(N_C2, N_P - N_C2), :] = jnp.zeros((N_P - N_C2, HID), f32)


def _tcl(hg1_ref, agg_ref, eps_ref, w1_ref, b1_ref, g_ref, be_ref,
         w2_ref, b2_ref, m_ref, w1o_ref, b1o_ref, w2o_ref, b2o_ref, out_ref):
    a = agg_ref[pl.ds(0, N_C2), :] + agg_ref[pl.ds(N_P, N_C2), :]
    h = (1.0 + eps_ref[...]) * hg1_ref[pl.ds(0, N_C2), :] + a
    hfin = jnp.maximum(
        _bn_mlp(h, w1_ref[...], b1_ref[...], g_ref[...], be_ref[...],
                w2_ref[...], b2_ref[...]), 0.0)
    h2m = jnp.mean(hfin, axis=0, keepdims=True)
    hh = jnp.concatenate(
        [m_ref[pl.ds(0, 1), :], m_ref[pl.ds(1, 1), :], h2m], axis=1)
    o = jnp.maximum(
        jnp.dot(hh, w1o_ref[...], preferred_element_type=f32) + b1o_ref[...],
        0.0)
    out_ref[...] = jnp.dot(o, w2o_ref[...],
                           preferred_element_type=f32) + b2o_ref[...]


def kernel(atom_feats, pimg, params, b1_src, b1_dst, b2_src, b2_dst,
           g1_src, g1_dst, i2_src, i2_dst):
    pb = params["bt"]
    ph = params["h2"]
    po = params["out"]
    (_sc_gat_a, _sc_gat_b, _sc_gin_bt, _sc_g1max, _sc_i2_384,
     _sc_i2_128) = _sc_kernels()

    Wg = jnp.stack([pb[r]["gat"]["W"] for r in range(2)])
    sel = ((jnp.arange(HID)[:, None] // HD)
           == jnp.arange(NH)[None, :]).astype(f32)
    Mlr = jnp.stack([
        jnp.concatenate(
            [sel * pb[r]["gat"]["attn_l"].reshape(-1)[:, None],
             sel * pb[r]["gat"]["attn_r"].reshape(-1)[:, None]], axis=1)
        for r in range(2)])
    e4 = ((jnp.arange(NH)[:, None])
          == (jnp.arange(HID)[None, :] // HD)).astype(f32)
    bias_g = jnp.stack([pb[r]["gat"]["bias"] for r in range(2)])

    def mlp_pack(ps):
        return [jnp.stack([p["mlp"][k] for p in ps]) for k in
                ("W1", "b1", "gamma", "beta", "W2", "b2")]

    gW1, gb1, gg, gbe, gW2, gb2 = mlp_pack([pb[0]["gin"], pb[1]["gin"]])
    geps = jnp.stack([pb[0]["gin"]["eps"], pb[1]["gin"]["eps"]]).reshape(2, 1)

    h0 = ph[0]["mlp"]
    hW1p = jnp.concatenate([h0["W1"], jnp.zeros((384 - 306, HID), f32)], axis=0)
    h1m = ph[1]["mlp"]

    def row(v):
        return v.reshape(1, -1)

    src_flat = jnp.concatenate([b1_src, b2_src]).astype(i32)
    dst_flat = jnp.concatenate([b1_dst, b2_dst]).astype(i32)
    g1d = g1_dst.astype(i32)
    pad = jnp.full((E_I2P - E_I2,), N_P - 1, i32)
    i2s = jnp.concatenate([i2_src.astype(i32), pad])
    i2d = jnp.concatenate([i2_dst.astype(i32), pad])

    z128 = jnp.zeros((640, HID), f32)
    zdn = jnp.zeros((NH * N_A,), f32)
    zmax = jnp.zeros((N_C2 * 16,), f32)
    z384 = jnp.zeros((384, HID), f32)
    z128b = jnp.zeros((128, HID), f32)

    feats, eler = _tc(_tca, (jax.ShapeDtypeStruct((2 * N_A, HID), f32),
                             jax.ShapeDtypeStruct((2 * N_A, 8), f32)))(
        atom_feats, Wg, Mlr)

    eeflat, dnm = _sc_gat_a(eler.reshape(-1), src_flat, dst_flat, zdn)
    rstu = _sc_gat_b(feats, eeflat, src_flat, dst_flat, z128)

    dn4 = dnm.reshape(NC, NS, NH, N_A)
    hgat = _tc(_tcd, jax.ShapeDtypeStruct((2 * N_A, HID), f32))(
        rstu, dn4[0], dn4[1], e4, bias_g)

    agg_bt = _sc_gin_bt(hgat, src_flat, dst_flat, z128)

    hA, means = _tc(_tcf, (jax.ShapeDtypeStruct((2 * N_A, HID), f32),
                           jax.ShapeDtypeStruct((2, HID), f32)))(
        hgat, agg_bt, geps, gW1, gb1, gg, gbe, gW2, gb2)

    hcb = _sc_g1max(hA, g1d, zmax)

    t384 = _tc(_tch, jax.ShapeDtypeStruct((N_P, 384), f32))(
        hcb.reshape(16 * N_C2, 16), pimg)

    agg1 = _sc_i2_384(t384.reshape(3 * N_P, HID), i2s, i2d,
                      z384).reshape(2 * N_P, 384)

    hg1 = _tc(_tcj, jax.ShapeDtypeStruct((N_P, HID), f32))(
        t384, agg1, ph[0]["eps"].reshape(1, 1), hW1p, row(h0["b1"]),
        row(h0["gamma"]), row(h0["beta"]), h0["W2"], row(h0["b2"]))

    agg2 = _sc_i2_128(hg1, i2s, i2d, z128b)

    out = _tc(_tcl, jax.ShapeDtypeStruct((1, HID), f32))(
        hg1, agg2, ph[1]["eps"].reshape(1, 1), h1m["W1"], row(h1m["b1"]),
        row(h1m["gamma"]), row(h1m["beta"]), h1m["W2"], row(h1m["b2"]),
        means, po["W1"], row(po["b1"]), po["W2"], row(po["b2"]))

    return out


# speedup vs baseline: 72.3028x; 1.7866x over previous
import functools

import jax
import jax.numpy as jnp
from jax import lax
from jax.experimental import pallas as pl
from jax.experimental.pallas import tpu as pltpu
from jax.experimental.pallas import tpu_sc as plsc

N_A = 10000
N_C2 = 2000
E_BT = 640000
E_I2 = 32000
HID = 128
NH = 4
HD = 32
PI = 50
NS = 16
NC = 2

f32 = jnp.float32
i32 = jnp.int32

_MESH = dict(core_axis_name="c", subcore_axis_name="s",
             num_cores=NC, num_subcores=NS)


def _full16(v):
    return jnp.full((16,), v, i32)


_CHB = 64
_NCHB = E_BT // NS // _CHB
_CHA = 400
_CHS = 128


def _make_sc_gat_a():
    @functools.partial(
        pl.kernel,
        out_type=(jax.ShapeDtypeStruct((2 * E_BT * NH,), f32),
                  jax.ShapeDtypeStruct((NC * NS * NH * N_A,), f32)),
        mesh=plsc.VectorSubcoreMesh(**_MESH),
        compiler_params=pltpu.CompilerParams(needs_layout_passes=False),
        scratch_types=[
            pltpu.VMEM((N_A * 8,), f32),
            pltpu.VMEM((NH * N_A,), f32),
            pltpu.VMEM((_CHA * NH,), f32),
            pltpu.VMEM((_CHA * NH,), f32),
            pltpu.VMEM((_CHA,), i32),
            pltpu.VMEM((_CHA,), i32),
            pltpu.VMEM((_CHA,), i32),
            pltpu.VMEM((_CHA,), i32),
            pltpu.SemaphoreType.DMA,
        ])
    def k(eler, srcs, dsts, zdn, ee_out, dnm,
          eler_t, denp, eeA, eeB, sA, dA, sB, dB, semW):
        c = lax.axis_index("c")
        s = lax.axis_index("s")
        pltpu.sync_copy(zdn, denp)
        pltpu.sync_copy(eler.at[pl.ds(pl.multiple_of(c * (N_A * 8), 8),
                                      N_A * 8)], eler_t)
        base = c * E_BT + s * (E_BT // NS)
        iota = lax.iota(i32, 16)

        def half(off, sb, db, eef):
            pltpu.sync_copy(srcs.at[pl.ds(off, _CHA)], sb)
            pltpu.sync_copy(dsts.at[pl.ds(off, _CHA)], db)

            def grp(j, carry2):
                sv8 = sb[pl.ds(j * 16, 16)] * 8
                dv = db[pl.ds(j * 16, 16)]
                dv8 = dv * 8
                rr = iota + j * 16
                for h in range(NH):
                    el = plsc.load_gather(eler_t, [sv8 + h])
                    er = plsc.load_gather(eler_t, [dv8 + (h + NH)])
                    e = el + er
                    e = jnp.where(e >= 0.0, e, 0.2 * e)
                    ee = jnp.exp(e)
                    plsc.addupdate_scatter(denp, [dv + h * N_A], ee)
                    plsc.store_scatter(eef, [rr * NH + h], ee)
                return carry2

            lax.fori_loop(0, _CHA // 16, grp, 0)
            return pltpu.async_copy(
                eef, ee_out.at[pl.ds(off * NH, _CHA * NH)], semW)

        def pair(p, carry):
            off = base + p * (2 * _CHA)
            cpA = half(off, sA, dA, eeA)
            cpB = half(off + _CHA, sB, dB, eeB)
            cpA.wait()
            cpB.wait()
            return carry

        lax.fori_loop(0, E_BT // NS // (2 * _CHA), pair, 0)
        pltpu.sync_copy(
            denp,
            dnm.at[pl.ds(pl.multiple_of((c * NS + s) * (NH * N_A), 8),
                         NH * N_A)])

    return k


def _make_sc_gat_b():
    @functools.partial(
        pl.kernel,
        out_type=jax.ShapeDtypeStruct((2 * N_A, HID), f32),
        mesh=plsc.VectorSubcoreMesh(**_MESH),
        compiler_params=pltpu.CompilerParams(needs_layout_passes=False),
        scratch_types=[
            pltpu.VMEM((_CHS, HID), f32),
            pltpu.VMEM((_CHS, HID), f32),
            pltpu.VMEM((_CHS * NH,), f32),
            pltpu.VMEM((_CHS * NH,), f32),
            pltpu.VMEM((_CHS,), i32),
            pltpu.VMEM((_CHS,), i32),
            pltpu.VMEM((_CHS,), i32),
            pltpu.VMEM((_CHS,), i32),
            pltpu.VMEM((_CHS,), i32),
            pltpu.VMEM((_CHS,), i32),
            pltpu.VMEM((64,), i32),
            pltpu.VMEM((64,), i32),
            pltpu.VMEM((64,), i32),
            pltpu.VMEM_SHARED((N_A, HID), f32),
            pltpu.SemaphoreType.DMA,
            pltpu.SemaphoreType.DMA,
        ])
    def k(feats, eehbm, srcs, dsts, z128, rstu,
          fbufA, fbufB, eeA, eeB, sA, dA, gA, sB, dB, gB, sT, dT, gT,
          acc, semG, semS):
        c = lax.axis_index("c")
        s = lax.axis_index("s")
        r0 = pl.multiple_of(s * 640, 8)
        g0 = pl.multiple_of(c * N_A + s * 640, 8)

        @pl.when(s < NS - 1)
        def _():
            pltpu.sync_copy(z128, acc.at[pl.ds(r0, 640)])

        @pl.when(s == NS - 1)
        def _():
            pltpu.sync_copy(z128.at[pl.ds(0, 400)], acc.at[pl.ds(r0, 400)])

        plsc.subcore_barrier()
        base = c * E_BT + s * (E_BT // NS)
        cbase = c * N_A

        def stage(off, sb, db, gb, fb, eef):
            pltpu.sync_copy(srcs.at[pl.ds(off, _CHS)], sb)
            pltpu.sync_copy(dsts.at[pl.ds(off, _CHS)], db)
            for j in range(_CHS // 16):
                gb[pl.ds(j * 16, 16)] = sb[pl.ds(j * 16, 16)] + cbase
            cp = pltpu.async_copy(feats.at[gb], fb, semG)
            pltpu.sync_copy(eehbm.at[pl.ds(off * NH, _CHS * NH)], eef)
            return cp

        def scale(fb, eef):
            def grp(j, carry2):
                for t in range(16):
                    ei = j * 16 + t
                    aa = [plsc.load_gather(eef, [_full16(ei * NH + h)])
                          for h in range(NH)]
                    for v in range(HID // 16):
                        fb[ei, pl.ds(v * 16, 16)] = (
                            fb[ei, pl.ds(v * 16, 16)] * aa[v * 16 // HD])
                return carry2

            lax.fori_loop(0, _CHS // 16, grp, 0)

        def pair(p, carry):
            off = base + p * (2 * _CHS)
            cpA = stage(off, sA, dA, gA, fbufA, eeA)
            cpB = stage(off + _CHS, sB, dB, gB, fbufB, eeB)
            cpA.wait()
            scale(fbufA, eeA)
            scA = pltpu.async_copy(fbufA, acc.at[dA], semS, add=True)
            cpB.wait()
            scale(fbufB, eeB)
            scB = pltpu.async_copy(fbufB, acc.at[dB], semS, add=True)
            scA.wait()
            scB.wait()
            return carry

        lax.fori_loop(0, E_BT // NS // (2 * _CHS), pair, 0)
        toff = base + (E_BT // NS // (2 * _CHS)) * (2 * _CHS)
        pltpu.sync_copy(srcs.at[pl.ds(toff, 64)], sT)
        pltpu.sync_copy(dsts.at[pl.ds(toff, 64)], dT)
        for j in range(4):
            gT[pl.ds(j * 16, 16)] = sT[pl.ds(j * 16, 16)] + cbase
        cp = pltpu.async_copy(feats.at[gT], fbufA.at[pl.ds(0, 64)], semG)
        pltpu.sync_copy(eehbm.at[pl.ds(toff * NH, 64 * NH)],
                        eeA.at[pl.ds(0, 64 * NH)])
        cp.wait()

        def tgrp(j, carry2):
            for t in range(16):
                ei = j * 16 + t
                aa = [plsc.load_gather(eeA, [_full16(ei * NH + h)])
                      for h in range(NH)]
                for v in range(HID // 16):
                    fbufA[ei, pl.ds(v * 16, 16)] = (
                        fbufA[ei, pl.ds(v * 16, 16)] * aa[v * 16 // HD])
            return carry2

        lax.fori_loop(0, 4, tgrp, 0)
        pltpu.sync_copy(fbufA.at[pl.ds(0, 64)], acc.at[dT], add=True)
        plsc.subcore_barrier()

        @pl.when(s < NS - 1)
        def _():
            pltpu.sync_copy(acc.at[pl.ds(r0, 640)], rstu.at[pl.ds(g0, 640)])

        @pl.when(s == NS - 1)
        def _():
            pltpu.sync_copy(acc.at[pl.ds(r0, 400)], rstu.at[pl.ds(g0, 400)])

    return k


def _make_sc_gin_bt():
    @functools.partial(
        pl.kernel,
        out_type=jax.ShapeDtypeStruct((2 * N_A, HID), f32),
        mesh=plsc.VectorSubcoreMesh(**_MESH),
        compiler_params=pltpu.CompilerParams(needs_layout_passes=False),
        scratch_types=[
            pltpu.VMEM((_CHS, HID), f32),
            pltpu.VMEM((_CHS, HID), f32),
            pltpu.VMEM((_CHS,), i32),
            pltpu.VMEM((_CHS,), i32),
            pltpu.VMEM((_CHS,), i32),
            pltpu.VMEM((_CHS,), i32),
            pltpu.VMEM((_CHS,), i32),
            pltpu.VMEM((_CHS,), i32),
            pltpu.VMEM((64,), i32),
            pltpu.VMEM((64,), i32),
            pltpu.VMEM((64,), i32),
            pltpu.VMEM_SHARED((N_A, HID), f32),
            pltpu.SemaphoreType.DMA,
            pltpu.SemaphoreType.DMA,
        ])
    def k(table, srcs, dsts, z128, agg,
          fbufA, fbufB, sA, dA, gA, sB, dB, gB, sT, dT, gT,
          acc, semG, semS):
        c = lax.axis_index("c")
        s = lax.axis_index("s")
        r0 = pl.multiple_of(s * 640, 8)
        g0 = pl.multiple_of(c * N_A + s * 640, 8)

        @pl.when(s < NS - 1)
        def _():
            pltpu.sync_copy(z128, acc.at[pl.ds(r0, 640)])

        @pl.when(s == NS - 1)
        def _():
            pltpu.sync_copy(z128.at[pl.ds(0, 400)], acc.at[pl.ds(r0, 400)])

        plsc.subcore_barrier()
        base = c * E_BT + s * (E_BT // NS)
        cbase = c * N_A

        def stage(off, sb, db, gb, fb):
            pltpu.sync_copy(srcs.at[pl.ds(off, _CHS)], sb)
            pltpu.sync_copy(dsts.at[pl.ds(off, _CHS)], db)
            for j in range(_CHS // 16):
                gb[pl.ds(j * 16, 16)] = sb[pl.ds(j * 16, 16)] + cbase
            return pltpu.async_copy(table.at[gb], fb, semG)

        def pair(p, carry):
            off = base + p * (2 * _CHS)
            cpA = stage(off, sA, dA, gA, fbufA)
            cpB = stage(off + _CHS, sB, dB, gB, fbufB)
            cpA.wait()
            scA = pltpu.async_copy(fbufA, acc.at[dA], semS, add=True)
            cpB.wait()
            scB = pltpu.async_copy(fbufB, acc.at[dB], semS, add=True)
            scA.wait()
            scB.wait()
            return carry

        lax.fori_loop(0, E_BT // NS // (2 * _CHS), pair, 0)
        toff = base + (E_BT // NS // (2 * _CHS)) * (2 * _CHS)
        pltpu.sync_copy(srcs.at[pl.ds(toff, 64)], sT)
        pltpu.sync_copy(dsts.at[pl.ds(toff, 64)], dT)
        for j in range(4):
            gT[pl.ds(j * 16, 16)] = sT[pl.ds(j * 16, 16)] + cbase
        pltpu.async_copy(table.at[gT], fbufA.at[pl.ds(0, 64)], semG).wait()
        pltpu.sync_copy(fbufA.at[pl.ds(0, 64)], acc.at[dT], add=True)
        plsc.subcore_barrier()

        @pl.when(s < NS - 1)
        def _():
            pltpu.sync_copy(acc.at[pl.ds(r0, 640)], agg.at[pl.ds(g0, 640)])

        @pl.when(s == NS - 1)
        def _():
            pltpu.sync_copy(acc.at[pl.ds(r0, 400)], agg.at[pl.ds(g0, 400)])

    return k


def _make_sc_g1max():
    @functools.partial(
        pl.kernel,
        out_type=jax.ShapeDtypeStruct((16 * N_C2 * 16,), f32),
        mesh=plsc.VectorSubcoreMesh(**_MESH),
        compiler_params=pltpu.CompilerParams(needs_layout_passes=False),
        scratch_types=[
            pltpu.VMEM((400, HID), f32),
            pltpu.VMEM((N_C2 * 16,), f32),
            pltpu.VMEM((400,), i32),
        ])
    def k(hA, g1dst, zmax, out, buf, acc, dstb):
        c = lax.axis_index("c")
        s = lax.axis_index("s")
        iota = lax.iota(i32, 16)

        @pl.when(s < 8)
        def _():
            pltpu.sync_copy(zmax, acc)
            cols = s * 16 + iota

            def rchunk(k2, carry):
                pltpu.sync_copy(
                    hA.at[pl.ds(pl.multiple_of(c * N_A + k2 * 400, 8), 400)],
                    buf)
                pltpu.sync_copy(g1dst.at[pl.ds(k2 * 400, 400)], dstb)

                def grp(g, carry2):
                    dv = dstb[pl.ds(g * 16, 16)]
                    nb = g * 16
                    for t in range(16):
                        di = _full16(dv[t]) * 16 + iota
                        v = plsc.load_gather(buf, [_full16(nb + t), cols])
                        cur = plsc.load_gather(acc, [di])
                        plsc.store_scatter(acc, [di], jnp.maximum(cur, v))
                    return carry2

                lax.fori_loop(0, 400 // 16, grp, 0)
                return carry

            lax.fori_loop(0, N_A // 400, rchunk, 0)
            pltpu.sync_copy(
                acc,
                out.at[pl.ds(pl.multiple_of((c * 8 + s) * (N_C2 * 16), 8),
                             N_C2 * 16)])

    return k


N_P = 2048
E_I2P = 32768


def _make_sc_i2(D):
    P = D // HID
    ept = E_I2P // (NC * NS)
    ch = 64

    @functools.partial(
        pl.kernel,
        out_type=jax.ShapeDtypeStruct((2 * P * N_P, HID), f32),
        mesh=plsc.VectorSubcoreMesh(**_MESH),
        compiler_params=pltpu.CompilerParams(needs_layout_passes=False),
        scratch_types=(
            [pltpu.VMEM((ch * P, HID), f32),
             pltpu.VMEM((ch,), i32),
             pltpu.VMEM((ch,), i32)]
            + [pltpu.VMEM((ch,), i32) for _ in range(2 * P)]
            + [pltpu.VMEM_SHARED((P * N_P, HID), f32),
               pltpu.SemaphoreType.DMA]),
    )
    def k(table, srcs, dsts, zrows, out, fbuf, sbuf, dbuf, *rest):
        idxs = rest[:2 * P]
        acc, sem = rest[2 * P], rest[2 * P + 1]
        c = lax.axis_index("c")
        s = lax.axis_index("s")
        rows = P * N_P // NS
        r0 = pl.multiple_of(s * rows, 8)
        g0 = pl.multiple_of(c * (P * N_P) + s * rows, 8)
        pltpu.sync_copy(zrows, acc.at[pl.ds(r0, rows)])
        plsc.subcore_barrier()
        base = (c * NS + s) * ept

        def chunk(kk, carry):
            off = base + kk * ch
            pltpu.sync_copy(srcs.at[pl.ds(off, ch)], sbuf)
            pltpu.sync_copy(dsts.at[pl.ds(off, ch)], dbuf)
            for p in range(P):
                for j in range(ch // 16):
                    sl = pl.ds(j * 16, 16)
                    idxs[p][sl] = sbuf[sl] * P + p
                    idxs[P + p][sl] = dbuf[sl] * P + p
            cps = [pltpu.async_copy(table.at[idxs[p]],
                                    fbuf.at[pl.ds(p * ch, ch)], sem)
                   for p in range(P)]
            for cp in cps:
                cp.wait()
            for p in range(P):
                pltpu.sync_copy(fbuf.at[pl.ds(p * ch, ch)],
                                acc.at[idxs[P + p]], add=True)
            return carry

        lax.fori_loop(0, ept // ch, chunk, 0)
        plsc.subcore_barrier()
        pltpu.sync_copy(acc.at[pl.ds(r0, rows)], out.at[pl.ds(g0, rows)])

    return k


@functools.lru_cache(maxsize=None)
def _sc_kernels():
    return (_make_sc_gat_a(), _make_sc_gat_b(), _make_sc_gin_bt(),
            _make_sc_g1max(), _make_sc_i2(384), _make_sc_i2(128))


def _tc(body, out_shape):
    return pl.pallas_call(body, out_shape=out_shape)


def _tca(x_ref, w_ref, m_ref, feats_ref, eler_ref):
    x = x_ref[...]
    for r in range(2):
        f = jnp.dot(x, w_ref[r], preferred_element_type=f32)
        feats_ref[pl.ds(r * N_A, N_A), :] = f
        eler_ref[pl.ds(r * N_A, N_A), :] = jnp.dot(
            f, m_ref[r], preferred_element_type=f32)


def _tcd(rstu_ref, d1_ref, d2_ref, e4_ref, bias_ref, out_ref):
    for r, dref in ((0, d1_ref), (1, d2_ref)):
        dn = jnp.sum(dref[...], axis=0)
        den = jnp.maximum(
            jax.lax.dot_general(dn, e4_ref[...], (((0,), (0,)), ((), ())),
                                preferred_element_type=f32), 1e-9)
        x = rstu_ref[pl.ds(r * N_A, N_A), :] / den
        out_ref[pl.ds(r * N_A, N_A), :] = jnp.maximum(
            x + bias_ref[pl.ds(r, 1), :], 0.0)


def _bn_mlp(h, w1, b1, g, be, w2, b2):
    h1 = jnp.dot(h, w1, preferred_element_type=f32) + b1
    mu = jnp.mean(h1, axis=0, keepdims=True)
    var = jnp.mean((h1 - mu) ** 2, axis=0, keepdims=True)
    hn = (h1 - mu) / jnp.sqrt(var + 1e-5) * g + be
    h2 = jnp.maximum(hn, 0.0)
    return jnp.dot(h2, w2, preferred_element_type=f32) + b2


def _tcf(hgat_ref, agg_ref, eps_ref, w1_ref, b1_ref, g_ref, be_ref,
         w2_ref, b2_ref, h_ref, m_ref):
    for r in range(2):
        sl = pl.ds(r * N_A, N_A)
        h = ((1.0 + eps_ref[pl.ds(r, 1), :]) * hgat_ref[sl, :]
             + agg_ref[sl, :])
        hr = jnp.maximum(
            _bn_mlp(h, w1_ref[r], b1_ref[pl.ds(r, 1), :], g_ref[pl.ds(r, 1), :],
                    be_ref[pl.ds(r, 1), :], w2_ref[r], b2_ref[pl.ds(r, 1), :]),
            0.0)
        h_ref[sl, :] = hr
        m_ref[pl.ds(r, 1), :] = jnp.mean(hr, axis=0, keepdims=True)


def _tch(hcb_ref, pimg_ref, out_ref):
    for i in range(16):
        off = 128 * (i // 8) + 16 * (i % 8)
        out_ref[pl.ds(0, N_C2), pl.ds(off, 16)] = hcb_ref[
            pl.ds(i * N_C2, N_C2), :]
    out_ref[pl.ds(0, N_C2), pl.ds(256, PI)] = pimg_ref[...]
    out_ref[pl.ds(0, N_C2), pl.ds(256 + PI, 384 - 256 - PI)] = jnp.zeros(
        (N_C2, 384 - 256 - PI), f32)
    out_ref[pl.ds(N_C2, N_P - N_C2), :] = jnp.zeros((N_P - N_C2, 384), f32)


def _tcj(t_ref, agg_ref, eps_ref, w1_ref, b1_ref, g_ref, be_ref,
         w2_ref, b2_ref, out_ref):
    a = agg_ref[pl.ds(0, N_C2), :] + agg_ref[pl.ds(N_P, N_C2), :]
    h = (1.0 + eps_ref[...]) * t_ref[pl.ds(0, N_C2), :] + a
    out_ref[pl.ds(0, N_C2), :] = jnp.maximum(
        _bn_mlp(h, w1_ref[...], b1_ref[...], g_ref[...], be_ref[...],
                w2_ref[...], b2_ref[...]), 0.0)
    out_ref[pl.ds(N_C2, N_P - N_C2), :] = jnp.zeros((N_P - N_C2, HID), f32)


def _tcl(hg1_ref, agg_ref, eps_ref, w1_ref, b1_ref, g_ref, be_ref,
         w2_ref, b2_ref, m_ref, w1o_ref, b1o_ref, w2o_ref, b2o_ref, out_ref):
    a = agg_ref[pl.ds(0, N_C2), :] + agg_ref[pl.ds(N_P, N_C2), :]
    h = (1.0 + eps_ref[...]) * hg1_ref[pl.ds(0, N_C2), :] + a
    hfin = jnp.maximum(
        _bn_mlp(h, w1_ref[...], b1_ref[...], g_ref[...], be_ref[...],
                w2_ref[...], b2_ref[...]), 0.0)
    h2m = jnp.mean(hfin, axis=0, keepdims=True)
    hh = jnp.concatenate(
        [m_ref[pl.ds(0, 1), :], m_ref[pl.ds(1, 1), :], h2m], axis=1)
    o = jnp.maximum(
        jnp.dot(hh, w1o_ref[...], preferred_element_type=f32) + b1o_ref[...],
        0.0)
    out_ref[...] = jnp.dot(o, w2o_ref[...],
                           preferred_element_type=f32) + b2o_ref[...]


def kernel(atom_feats, pimg, params, b1_src, b1_dst, b2_src, b2_dst,
           g1_src, g1_dst, i2_src, i2_dst):
    pb = params["bt"]
    ph = params["h2"]
    po = params["out"]
    (_sc_gat_a, _sc_gat_b, _sc_gin_bt, _sc_g1max, _sc_i2_384,
     _sc_i2_128) = _sc_kernels()

    Wg = jnp.stack([pb[r]["gat"]["W"] for r in range(2)])
    sel = ((jnp.arange(HID)[:, None] // HD)
           == jnp.arange(NH)[None, :]).astype(f32)
    Mlr = jnp.stack([
        jnp.concatenate(
            [sel * pb[r]["gat"]["attn_l"].reshape(-1)[:, None],
             sel * pb[r]["gat"]["attn_r"].reshape(-1)[:, None]], axis=1)
        for r in range(2)])
    e4 = ((jnp.arange(NH)[:, None])
          == (jnp.arange(HID)[None, :] // HD)).astype(f32)
    bias_g = jnp.stack([pb[r]["gat"]["bias"] for r in range(2)])

    def mlp_pack(ps):
        return [jnp.stack([p["mlp"][k] for p in ps]) for k in
                ("W1", "b1", "gamma", "beta", "W2", "b2")]

    gW1, gb1, gg, gbe, gW2, gb2 = mlp_pack([pb[0]["gin"], pb[1]["gin"]])
    geps = jnp.stack([pb[0]["gin"]["eps"], pb[1]["gin"]["eps"]]).reshape(2, 1)

    h0 = ph[0]["mlp"]
    hW1p = jnp.concatenate([h0["W1"], jnp.zeros((384 - 306, HID), f32)], axis=0)
    h1m = ph[1]["mlp"]

    def row(v):
        return v.reshape(1, -1)

    src_flat = jnp.concatenate([b1_src, b2_src]).astype(i32)
    dst_flat = jnp.concatenate([b1_dst, b2_dst]).astype(i32)
    g1d = g1_dst.astype(i32)
    pad = jnp.full((E_I2P - E_I2,), N_P - 1, i32)
    i2s = jnp.concatenate([i2_src.astype(i32), pad])
    i2d = jnp.concatenate([i2_dst.astype(i32), pad])

    z128 = jnp.zeros((640, HID), f32)
    zdn = jnp.zeros((NH * N_A,), f32)
    zmax = jnp.zeros((N_C2 * 16,), f32)
    z384 = jnp.zeros((384, HID), f32)
    z128b = jnp.zeros((128, HID), f32)

    feats, eler = _tc(_tca, (jax.ShapeDtypeStruct((2 * N_A, HID), f32),
                             jax.ShapeDtypeStruct((2 * N_A, 8), f32)))(
        atom_feats, Wg, Mlr)

    eeflat, dnm = _sc_gat_a(eler.reshape(-1), src_flat, dst_flat, zdn)
    rstu = _sc_gat_b(feats, eeflat, src_flat, dst_flat, z128)

    dn4 = dnm.reshape(NC, NS, NH, N_A)
    hgat = _tc(_tcd, jax.ShapeDtypeStruct((2 * N_A, HID), f32))(
        rstu, dn4[0], dn4[1], e4, bias_g)

    agg_bt = _sc_gin_bt(hgat, src_flat, dst_flat, z128)

    hA, means = _tc(_tcf, (jax.ShapeDtypeStruct((2 * N_A, HID), f32),
                           jax.ShapeDtypeStruct((2, HID), f32)))(
        hgat, agg_bt, geps, gW1, gb1, gg, gbe, gW2, gb2)

    hcb = _sc_g1max(hA, g1d, zmax)

    t384 = _tc(_tch, jax.ShapeDtypeStruct((N_P, 384), f32))(
        hcb.reshape(16 * N_C2, 16), pimg)

    agg1 = _sc_i2_384(t384.reshape(3 * N_P, HID), i2s, i2d,
                      z384).reshape(2 * N_P, 384)

    hg1 = _tc(_tcj, jax.ShapeDtypeStruct((N_P, HID), f32))(
        t384, agg1, ph[0]["eps"].reshape(1, 1), hW1p, row(h0["b1"]),
        row(h0["gamma"]), row(h0["beta"]), h0["W2"], row(h0["b2"]))

    agg2 = _sc_i2_128(hg1, i2s, i2d, z128b)

    out = _tc(_tcl, jax.ShapeDtypeStruct((1, HID), f32))(
        hg1, agg2, ph[1]["eps"].reshape(1, 1), h1m["W1"], row(h1m["b1"]),
        row(h1m["gamma"]), row(h1m["beta"]), h1m["W2"], row(h1m["b2"]),
        means, po["W1"], row(po["b1"]), po["W2"], row(po["b2"]))

    return out


# speedup vs baseline: 90.9052x; 1.2573x over previous
import functools

import jax
import jax.numpy as jnp
from jax import lax
from jax.experimental import pallas as pl
from jax.experimental.pallas import tpu as pltpu
from jax.experimental.pallas import tpu_sc as plsc

N_A = 10000
N_C2 = 2000
E_BT = 640000
E_I2 = 32000
HID = 128
NH = 4
HD = 32
PI = 50
NS = 16
NC = 2

f32 = jnp.float32
i32 = jnp.int32

_MESH = dict(core_axis_name="c", subcore_axis_name="s",
             num_cores=NC, num_subcores=NS)


def _full16(v):
    return jnp.full((16,), v, i32)


_CHB = 64
_NCHB = E_BT // NS // _CHB
_CHA = 400
_CHS = 128
_BLK = 1024


def _make_sc_gat_a():
    @functools.partial(
        pl.kernel,
        out_type=(jax.ShapeDtypeStruct((2 * E_BT * NH,), f32),
                  jax.ShapeDtypeStruct((NC * NS * NH * N_A,), f32)),
        mesh=plsc.VectorSubcoreMesh(**_MESH),
        compiler_params=pltpu.CompilerParams(needs_layout_passes=False),
        scratch_types=[
            pltpu.VMEM((N_A * 8,), f32),
            pltpu.VMEM((NH * N_A,), f32),
            pltpu.VMEM((_CHA * NH,), f32),
            pltpu.VMEM((_CHA * NH,), f32),
            pltpu.VMEM((_CHA,), i32),
            pltpu.VMEM((_CHA,), i32),
            pltpu.VMEM((_CHA,), i32),
            pltpu.VMEM((_CHA,), i32),
            pltpu.SemaphoreType.DMA,
        ])
    def k(eler, srcs, dsts, zdn, ee_out, dnm,
          eler_t, denp, eeA, eeB, sA, dA, sB, dB, semW):
        c = lax.axis_index("c")
        s = lax.axis_index("s")
        pltpu.sync_copy(zdn, denp)
        pltpu.sync_copy(eler.at[pl.ds(pl.multiple_of(c * (N_A * 8), 8),
                                      N_A * 8)], eler_t)
        base = c * E_BT + s * (E_BT // NS)
        iota = lax.iota(i32, 16)

        def half(off, sb, db, eef):
            pltpu.sync_copy(srcs.at[pl.ds(off, _CHA)], sb)
            pltpu.sync_copy(dsts.at[pl.ds(off, _CHA)], db)

            def grp(j, carry2):
                sv8 = sb[pl.ds(j * 16, 16)] * 8
                dv = db[pl.ds(j * 16, 16)]
                dv8 = dv * 8
                rr = iota + j * 16
                for h in range(NH):
                    el = plsc.load_gather(eler_t, [sv8 + h])
                    er = plsc.load_gather(eler_t, [dv8 + (h + NH)])
                    e = el + er
                    e = jnp.where(e >= 0.0, e, 0.2 * e)
                    ee = jnp.exp(e)
                    plsc.addupdate_scatter(denp, [dv + h * N_A], ee)
                    plsc.store_scatter(eef, [rr * NH + h], ee)
                return carry2

            lax.fori_loop(0, _CHA // 16, grp, 0)
            return pltpu.async_copy(
                eef, ee_out.at[pl.ds(off * NH, _CHA * NH)], semW)

        def pair(p, carry):
            off = base + p * (2 * _CHA)
            cpA = half(off, sA, dA, eeA)
            cpB = half(off + _CHA, sB, dB, eeB)
            cpA.wait()
            cpB.wait()
            return carry

        lax.fori_loop(0, E_BT // NS // (2 * _CHA), pair, 0)
        pltpu.sync_copy(
            denp,
            dnm.at[pl.ds(pl.multiple_of((c * NS + s) * (NH * N_A), 8),
                         NH * N_A)])

    return k


def _make_sc_gat_b():
    @functools.partial(
        pl.kernel,
        out_type=jax.ShapeDtypeStruct((2 * N_A, HID), f32),
        mesh=plsc.VectorSubcoreMesh(**_MESH),
        compiler_params=pltpu.CompilerParams(needs_layout_passes=False),
        scratch_types=[
            pltpu.VMEM((_CHS, HID), f32),
            pltpu.VMEM((_CHS, HID), f32),
            pltpu.VMEM((_BLK * NH,), f32),
            pltpu.VMEM((_BLK,), i32),
            pltpu.VMEM((_BLK // _CHS, _CHS), i32),
            pltpu.VMEM((_BLK // _CHS, _CHS), i32),
            pltpu.VMEM_SHARED((N_A, HID), f32),
            pltpu.SemaphoreType.DMA,
            pltpu.SemaphoreType.DMA,
        ])
    def k(feats, eehbm, srcs, dsts2, z128, rstu,
          fbufA, fbufB, eeblk, sblk, gblk, dblk,
          acc, semG, semS):
        c = lax.axis_index("c")
        s = lax.axis_index("s")
        r0 = pl.multiple_of(s * 640, 8)
        g0 = pl.multiple_of(c * N_A + s * 640, 8)

        @pl.when(s < NS - 1)
        def _():
            pltpu.sync_copy(z128, acc.at[pl.ds(r0, 640)])

        @pl.when(s == NS - 1)
        def _():
            pltpu.sync_copy(z128.at[pl.ds(0, 400)], acc.at[pl.ds(r0, 400)])

        plsc.subcore_barrier()
        base = c * E_BT + jnp.where(s == 0, 0, 40960 + (s - 1) * 39936)
        nblk = jnp.where(s == 0, 40, 39)
        cbase = c * N_A
        bufs = (fbufA, fbufB)

        def block(b, carry):
            off = base + b * _BLK
            pltpu.sync_copy(srcs.at[pl.ds(off, _BLK)], sblk)
            pltpu.sync_copy(
                dsts2.at[pl.ds(pl.multiple_of(off // _CHS, 8),
                               _BLK // _CHS)], dblk)
            pltpu.sync_copy(eehbm.at[pl.ds(off * NH, _BLK * NH)], eeblk)
            for j in range(_BLK // 16):
                gblk[j // 8, pl.ds((j % 8) * 16, 16)] = (
                    sblk[pl.ds(j * 16, 16)] + cbase)

            def scale(fb, jj):
                def grp(g, carry2):
                    for t in range(16):
                        ei = g * 16 + t
                        eoff = (jj * _CHS + ei) * NH
                        aa = [plsc.load_gather(eeblk, [_full16(eoff + h)])
                              for h in range(NH)]
                        for v in range(HID // 16):
                            fb[ei, pl.ds(v * 16, 16)] = (
                                fb[ei, pl.ds(v * 16, 16)] * aa[v * 16 // HD])
                    return carry2

                lax.fori_loop(0, _CHS // 16, grp, 0)

            cps = [None] * 8
            scs = [None] * 8
            cps[0] = pltpu.async_copy(feats.at[gblk.at[0]], bufs[0], semG)
            for j in range(8):
                if j + 1 < 8:
                    if j >= 1:
                        scs[j - 1].wait()
                    cps[j + 1] = pltpu.async_copy(
                        feats.at[gblk.at[j + 1]], bufs[(j + 1) % 2], semG)
                cps[j].wait()
                scale(bufs[j % 2], j)
                scs[j] = pltpu.async_copy(bufs[j % 2], acc.at[dblk.at[j]],
                                          semS, add=True)
            scs[6].wait()
            scs[7].wait()
            return carry

        lax.fori_loop(0, nblk, block, 0)
        plsc.subcore_barrier()

        @pl.when(s < NS - 1)
        def _():
            pltpu.sync_copy(acc.at[pl.ds(r0, 640)], rstu.at[pl.ds(g0, 640)])

        @pl.when(s == NS - 1)
        def _():
            pltpu.sync_copy(acc.at[pl.ds(r0, 400)], rstu.at[pl.ds(g0, 400)])

    return k


def _make_sc_gin_bt():
    @functools.partial(
        pl.kernel,
        out_type=jax.ShapeDtypeStruct((2 * N_A, HID), f32),
        mesh=plsc.VectorSubcoreMesh(**_MESH),
        compiler_params=pltpu.CompilerParams(needs_layout_passes=False),
        scratch_types=[
            pltpu.VMEM((_CHS, HID), f32),
            pltpu.VMEM((_CHS, HID), f32),
            pltpu.VMEM((_BLK,), i32),
            pltpu.VMEM((_BLK // _CHS, _CHS), i32),
            pltpu.VMEM((_BLK // _CHS, _CHS), i32),
            pltpu.VMEM_SHARED((N_A, HID), f32),
            pltpu.SemaphoreType.DMA,
            pltpu.SemaphoreType.DMA,
        ])
    def k(table, srcs, dsts2, z128, agg,
          fbufA, fbufB, sblk, gblk, dblk, acc, semG, semS):
        c = lax.axis_index("c")
        s = lax.axis_index("s")
        r0 = pl.multiple_of(s * 640, 8)
        g0 = pl.multiple_of(c * N_A + s * 640, 8)

        @pl.when(s < NS - 1)
        def _():
            pltpu.sync_copy(z128, acc.at[pl.ds(r0, 640)])

        @pl.when(s == NS - 1)
        def _():
            pltpu.sync_copy(z128.at[pl.ds(0, 400)], acc.at[pl.ds(r0, 400)])

        plsc.subcore_barrier()
        base = c * E_BT + jnp.where(s == 0, 0, 40960 + (s - 1) * 39936)
        nblk = jnp.where(s == 0, 40, 39)
        cbase = c * N_A
        bufs = (fbufA, fbufB)

        def block(b, carry):
            off = base + b * _BLK
            pltpu.sync_copy(srcs.at[pl.ds(off, _BLK)], sblk)
            pltpu.sync_copy(
                dsts2.at[pl.ds(pl.multiple_of(off // _CHS, 8),
                               _BLK // _CHS)], dblk)
            for j in range(_BLK // 16):
                gblk[j // 8, pl.ds((j % 8) * 16, 16)] = (
                    sblk[pl.ds(j * 16, 16)] + cbase)
            cps = [None] * 8
            scs = [None] * 8
            cps[0] = pltpu.async_copy(table.at[gblk.at[0]], bufs[0], semG)
            for j in range(8):
                if j + 1 < 8:
                    if j >= 1:
                        scs[j - 1].wait()
                    cps[j + 1] = pltpu.async_copy(
                        table.at[gblk.at[j + 1]], bufs[(j + 1) % 2], semG)
                cps[j].wait()
                scs[j] = pltpu.async_copy(bufs[j % 2], acc.at[dblk.at[j]],
                                          semS, add=True)
            scs[6].wait()
            scs[7].wait()
            return carry

        lax.fori_loop(0, nblk, block, 0)
        plsc.subcore_barrier()

        @pl.when(s < NS - 1)
        def _():
            pltpu.sync_copy(acc.at[pl.ds(r0, 640)], agg.at[pl.ds(g0, 640)])

        @pl.when(s == NS - 1)
        def _():
            pltpu.sync_copy(acc.at[pl.ds(r0, 400)], agg.at[pl.ds(g0, 400)])

    return k


def _make_sc_g1max():
    @functools.partial(
        pl.kernel,
        out_type=jax.ShapeDtypeStruct((16 * N_C2 * 16,), f32),
        mesh=plsc.VectorSubcoreMesh(**_MESH),
        compiler_params=pltpu.CompilerParams(needs_layout_passes=False),
        scratch_types=[
            pltpu.VMEM((400, HID), f32),
            pltpu.VMEM((N_C2 * 16,), f32),
            pltpu.VMEM((400,), i32),
        ])
    def k(hA, g1dst, zmax, out, buf, acc, dstb):
        c = lax.axis_index("c")
        s = lax.axis_index("s")
        iota = lax.iota(i32, 16)

        @pl.when(s < 8)
        def _():
            pltpu.sync_copy(zmax, acc)
            cols = s * 16 + iota

            def rchunk(k2, carry):
                pltpu.sync_copy(
                    hA.at[pl.ds(pl.multiple_of(c * N_A + k2 * 400, 8), 400)],
                    buf)
                pltpu.sync_copy(g1dst.at[pl.ds(k2 * 400, 400)], dstb)

                def grp(g, carry2):
                    dv = dstb[pl.ds(g * 16, 16)]
                    nb = g * 16
                    for t in range(16):
                        di = _full16(dv[t]) * 16 + iota
                        v = plsc.load_gather(buf, [_full16(nb + t), cols])
                        cur = plsc.load_gather(acc, [di])
                        plsc.store_scatter(acc, [di], jnp.maximum(cur, v))
                    return carry2

                lax.fori_loop(0, 400 // 16, grp, 0)
                return carry

            lax.fori_loop(0, N_A // 400, rchunk, 0)
            pltpu.sync_copy(
                acc,
                out.at[pl.ds(pl.multiple_of((c * 8 + s) * (N_C2 * 16), 8),
                             N_C2 * 16)])

    return k


N_P = 2048
E_I2P = 32768


def _make_sc_i2(D):
    P = D // HID
    ept = E_I2P // (NC * NS)
    ch = 64

    @functools.partial(
        pl.kernel,
        out_type=jax.ShapeDtypeStruct((2 * P * N_P, HID), f32),
        mesh=plsc.VectorSubcoreMesh(**_MESH),
        compiler_params=pltpu.CompilerParams(needs_layout_passes=False),
        scratch_types=(
            [pltpu.VMEM((ch * P, HID), f32),
             pltpu.VMEM((ch,), i32),
             pltpu.VMEM((ch,), i32)]
            + [pltpu.VMEM((ch,), i32) for _ in range(2 * P)]
            + [pltpu.VMEM_SHARED((P * N_P, HID), f32),
               pltpu.SemaphoreType.DMA]),
    )
    def k(table, srcs, dsts, zrows, out, fbuf, sbuf, dbuf, *rest):
        idxs = rest[:2 * P]
        acc, sem = rest[2 * P], rest[2 * P + 1]
        c = lax.axis_index("c")
        s = lax.axis_index("s")
        rows = P * N_P // NS
        r0 = pl.multiple_of(s * rows, 8)
        g0 = pl.multiple_of(c * (P * N_P) + s * rows, 8)
        pltpu.sync_copy(zrows, acc.at[pl.ds(r0, rows)])
        plsc.subcore_barrier()
        base = (c * NS + s) * ept

        def chunk(kk, carry):
            off = base + kk * ch
            pltpu.sync_copy(srcs.at[pl.ds(off, ch)], sbuf)
            pltpu.sync_copy(dsts.at[pl.ds(off, ch)], dbuf)
            for p in range(P):
                for j in range(ch // 16):
                    sl = pl.ds(j * 16, 16)
                    idxs[p][sl] = sbuf[sl] * P + p
                    idxs[P + p][sl] = dbuf[sl] * P + p
            cps = [pltpu.async_copy(table.at[idxs[p]],
                                    fbuf.at[pl.ds(p * ch, ch)], sem)
                   for p in range(P)]
            for cp in cps:
                cp.wait()
            for p in range(P):
                pltpu.sync_copy(fbuf.at[pl.ds(p * ch, ch)],
                                acc.at[idxs[P + p]], add=True)
            return carry

        lax.fori_loop(0, ept // ch, chunk, 0)
        plsc.subcore_barrier()
        pltpu.sync_copy(acc.at[pl.ds(r0, rows)], out.at[pl.ds(g0, rows)])

    return k


@functools.lru_cache(maxsize=None)
def _sc_kernels():
    return (_make_sc_gat_a(), _make_sc_gat_b(), _make_sc_gin_bt(),
            _make_sc_g1max(), _make_sc_i2(384), _make_sc_i2(128))


def _tc(body, out_shape):
    return pl.pallas_call(body, out_shape=out_shape)


def _tca(x_ref, w_ref, m_ref, feats_ref, eler_ref):
    x = x_ref[...]
    for r in range(2):
        f = jnp.dot(x, w_ref[r], preferred_element_type=f32)
        feats_ref[pl.ds(r * N_A, N_A), :] = f
        eler_ref[pl.ds(r * N_A, N_A), :] = jnp.dot(
            f, m_ref[r], preferred_element_type=f32)


def _tcd(rstu_ref, d1_ref, d2_ref, e4_ref, bias_ref, out_ref):
    for r, dref in ((0, d1_ref), (1, d2_ref)):
        dn = jnp.sum(dref[...], axis=0)
        den = jnp.maximum(
            jax.lax.dot_general(dn, e4_ref[...], (((0,), (0,)), ((), ())),
                                preferred_element_type=f32), 1e-9)
        x = rstu_ref[pl.ds(r * N_A, N_A), :] / den
        out_ref[pl.ds(r * N_A, N_A), :] = jnp.maximum(
            x + bias_ref[pl.ds(r, 1), :], 0.0)


def _bn_mlp(h, w1, b1, g, be, w2, b2):
    h1 = jnp.dot(h, w1, preferred_element_type=f32) + b1
    mu = jnp.mean(h1, axis=0, keepdims=True)
    var = jnp.mean((h1 - mu) ** 2, axis=0, keepdims=True)
    hn = (h1 - mu) / jnp.sqrt(var + 1e-5) * g + be
    h2 = jnp.maximum(hn, 0.0)
    return jnp.dot(h2, w2, preferred_element_type=f32) + b2


def _tcf(hgat_ref, agg_ref, eps_ref, w1_ref, b1_ref, g_ref, be_ref,
         w2_ref, b2_ref, h_ref, m_ref):
    for r in range(2):
        sl = pl.ds(r * N_A, N_A)
        h = ((1.0 + eps_ref[pl.ds(r, 1), :]) * hgat_ref[sl, :]
             + agg_ref[sl, :])
        hr = jnp.maximum(
            _bn_mlp(h, w1_ref[r], b1_ref[pl.ds(r, 1), :], g_ref[pl.ds(r, 1), :],
                    be_ref[pl.ds(r, 1), :], w2_ref[r], b2_ref[pl.ds(r, 1), :]),
            0.0)
        h_ref[sl, :] = hr
        m_ref[pl.ds(r, 1), :] = jnp.mean(hr, axis=0, keepdims=True)


def _tch(hcb_ref, pimg_ref, out_ref):
    for i in range(16):
        off = 128 * (i // 8) + 16 * (i % 8)
        out_ref[pl.ds(0, N_C2), pl.ds(off, 16)] = hcb_ref[
            pl.ds(i * N_C2, N_C2), :]
    out_ref[pl.ds(0, N_C2), pl.ds(256, PI)] = pimg_ref[...]
    out_ref[pl.ds(0, N_C2), pl.ds(256 + PI, 384 - 256 - PI)] = jnp.zeros(
        (N_C2, 384 - 256 - PI), f32)
    out_ref[pl.ds(N_C2, N_P - N_C2), :] = jnp.zeros((N_P - N_C2, 384), f32)


def _tcj(t_ref, agg_ref, eps_ref, w1_ref, b1_ref, g_ref, be_ref,
         w2_ref, b2_ref, out_ref):
    a = agg_ref[pl.ds(0, N_C2), :] + agg_ref[pl.ds(N_P, N_C2), :]
    h = (1.0 + eps_ref[...]) * t_ref[pl.ds(0, N_C2), :] + a
    out_ref[pl.ds(0, N_C2), :] = jnp.maximum(
        _bn_mlp(h, w1_ref[...], b1_ref[...], g_ref[...], be_ref[...],
                w2_ref[...], b2_ref[...]), 0.0)
    out_ref[pl.ds(N_C2, N_P - N_C2), :] = jnp.zeros((N_P - N_C2, HID), f32)


def _tcl(hg1_ref, agg_ref, eps_ref, w1_ref, b1_ref, g_ref, be_ref,
         w2_ref, b2_ref, m_ref, w1o_ref, b1o_ref, w2o_ref, b2o_ref, out_ref):
    a = agg_ref[pl.ds(0, N_C2), :] + agg_ref[pl.ds(N_P, N_C2), :]
    h = (1.0 + eps_ref[...]) * hg1_ref[pl.ds(0, N_C2), :] + a
    hfin = jnp.maximum(
        _bn_mlp(h, w1_ref[...], b1_ref[...], g_ref[...], be_ref[...],
                w2_ref[...], b2_ref[...]), 0.0)
    h2m = jnp.mean(hfin, axis=0, keepdims=True)
    hh = jnp.concatenate(
        [m_ref[pl.ds(0, 1), :], m_ref[pl.ds(1, 1), :], h2m], axis=1)
    o = jnp.maximum(
        jnp.dot(hh, w1o_ref[...], preferred_element_type=f32) + b1o_ref[...],
        0.0)
    out_ref[...] = jnp.dot(o, w2o_ref[...],
                           preferred_element_type=f32) + b2o_ref[...]


def kernel(atom_feats, pimg, params, b1_src, b1_dst, b2_src, b2_dst,
           g1_src, g1_dst, i2_src, i2_dst):
    pb = params["bt"]
    ph = params["h2"]
    po = params["out"]
    (_sc_gat_a, _sc_gat_b, _sc_gin_bt, _sc_g1max, _sc_i2_384,
     _sc_i2_128) = _sc_kernels()

    Wg = jnp.stack([pb[r]["gat"]["W"] for r in range(2)])
    sel = ((jnp.arange(HID)[:, None] // HD)
           == jnp.arange(NH)[None, :]).astype(f32)
    Mlr = jnp.stack([
        jnp.concatenate(
            [sel * pb[r]["gat"]["attn_l"].reshape(-1)[:, None],
             sel * pb[r]["gat"]["attn_r"].reshape(-1)[:, None]], axis=1)
        for r in range(2)])
    e4 = ((jnp.arange(NH)[:, None])
          == (jnp.arange(HID)[None, :] // HD)).astype(f32)
    bias_g = jnp.stack([pb[r]["gat"]["bias"] for r in range(2)])

    def mlp_pack(ps):
        return [jnp.stack([p["mlp"][k] for p in ps]) for k in
                ("W1", "b1", "gamma", "beta", "W2", "b2")]

    gW1, gb1, gg, gbe, gW2, gb2 = mlp_pack([pb[0]["gin"], pb[1]["gin"]])
    geps = jnp.stack([pb[0]["gin"]["eps"], pb[1]["gin"]["eps"]]).reshape(2, 1)

    h0 = ph[0]["mlp"]
    hW1p = jnp.concatenate([h0["W1"], jnp.zeros((384 - 306, HID), f32)], axis=0)
    h1m = ph[1]["mlp"]

    def row(v):
        return v.reshape(1, -1)

    src_flat = jnp.concatenate([b1_src, b2_src]).astype(i32)
    dst_flat = jnp.concatenate([b1_dst, b2_dst]).astype(i32)
    g1d = g1_dst.astype(i32)
    pad = jnp.full((E_I2P - E_I2,), N_P - 1, i32)
    i2s = jnp.concatenate([i2_src.astype(i32), pad])
    i2d = jnp.concatenate([i2_dst.astype(i32), pad])

    z128 = jnp.zeros((640, HID), f32)
    zdn = jnp.zeros((NH * N_A,), f32)
    zmax = jnp.zeros((N_C2 * 16,), f32)
    z384 = jnp.zeros((384, HID), f32)
    z128b = jnp.zeros((128, HID), f32)

    feats, eler = _tc(_tca, (jax.ShapeDtypeStruct((2 * N_A, HID), f32),
                             jax.ShapeDtypeStruct((2 * N_A, 8), f32)))(
        atom_feats, Wg, Mlr)

    eeflat, dnm = _sc_gat_a(eler.reshape(-1), src_flat, dst_flat, zdn)
    dst2 = dst_flat.reshape(-1, _CHS)
    rstu = _sc_gat_b(feats, eeflat, src_flat, dst2, z128)

    dn4 = dnm.reshape(NC, NS, NH, N_A)
    hgat = _tc(_tcd, jax.ShapeDtypeStruct((2 * N_A, HID), f32))(
        rstu, dn4[0], dn4[1], e4, bias_g)

    agg_bt = _sc_gin_bt(hgat, src_flat, dst2, z128)

    hA, means = _tc(_tcf, (jax.ShapeDtypeStruct((2 * N_A, HID), f32),
                           jax.ShapeDtypeStruct((2, HID), f32)))(
        hgat, agg_bt, geps, gW1, gb1, gg, gbe, gW2, gb2)

    hcb = _sc_g1max(hA, g1d, zmax)

    t384 = _tc(_tch, jax.ShapeDtypeStruct((N_P, 384), f32))(
        hcb.reshape(16 * N_C2, 16), pimg)

    agg1 = _sc_i2_384(t384.reshape(3 * N_P, HID), i2s, i2d,
                      z384).reshape(2 * N_P, 384)

    hg1 = _tc(_tcj, jax.ShapeDtypeStruct((N_P, HID), f32))(
        t384, agg1, ph[0]["eps"].reshape(1, 1), hW1p, row(h0["b1"]),
        row(h0["gamma"]), row(h0["beta"]), h0["W2"], row(h0["b2"]))

    agg2 = _sc_i2_128(hg1, i2s, i2d, z128b)

    out = _tc(_tcl, jax.ShapeDtypeStruct((1, HID), f32))(
        hg1, agg2, ph[1]["eps"].reshape(1, 1), h1m["W1"], row(h1m["b1"]),
        row(h1m["gamma"]), row(h1m["beta"]), h1m["W2"], row(h1m["b2"]),
        means, po["W1"], row(po["b1"]), po["W2"], row(po["b2"]))

    return out


# speedup vs baseline: 93.1812x; 1.0250x over previous
import functools

import jax
import jax.numpy as jnp
from jax import lax
from jax.experimental import pallas as pl
from jax.experimental.pallas import tpu as pltpu
from jax.experimental.pallas import tpu_sc as plsc

N_A = 10000
N_C2 = 2000
E_BT = 640000
E_I2 = 32000
HID = 128
NH = 4
HD = 32
PI = 50
NS = 16
NC = 2

f32 = jnp.float32
i32 = jnp.int32

_MESH = dict(core_axis_name="c", subcore_axis_name="s",
             num_cores=NC, num_subcores=NS)


def _full16(v):
    return jnp.full((16,), v, i32)


_CHB = 64
_NCHB = E_BT // NS // _CHB
_CHA = 800
_CHS = 128
_BLK = 1024


def _make_sc_gat_a():
    @functools.partial(
        pl.kernel,
        out_type=(jax.ShapeDtypeStruct((2 * E_BT * NH,), f32),
                  jax.ShapeDtypeStruct((NC * NS * NH * N_A,), f32)),
        mesh=plsc.VectorSubcoreMesh(**_MESH),
        compiler_params=pltpu.CompilerParams(needs_layout_passes=False),
        scratch_types=[
            pltpu.VMEM((N_A * 8,), f32),
            pltpu.VMEM((NH * N_A,), f32),
            pltpu.VMEM((_CHA * NH,), f32),
            pltpu.VMEM((_CHA * NH,), f32),
            pltpu.VMEM((_CHA,), i32),
            pltpu.VMEM((_CHA,), i32),
            pltpu.VMEM((_CHA,), i32),
            pltpu.VMEM((_CHA,), i32),
            pltpu.SemaphoreType.DMA,
        ])
    def k(eler, srcs, dsts, zdn, ee_out, dnm,
          eler_t, denp, eeA, eeB, sA, dA, sB, dB, semW):
        c = lax.axis_index("c")
        s = lax.axis_index("s")
        pltpu.sync_copy(zdn, denp)
        pltpu.sync_copy(eler.at[pl.ds(pl.multiple_of(c * (N_A * 8), 8),
                                      N_A * 8)], eler_t)
        base = c * E_BT + s * (E_BT // NS)
        iota = lax.iota(i32, 16)

        def half(off, sb, db, eef):
            pltpu.sync_copy(srcs.at[pl.ds(off, _CHA)], sb)
            pltpu.sync_copy(dsts.at[pl.ds(off, _CHA)], db)

            def grp(j, carry2):
                sv8 = sb[pl.ds(j * 16, 16)] * 8
                dv = db[pl.ds(j * 16, 16)]
                dv8 = dv * 8
                rr = iota + j * 16
                for h in range(NH):
                    el = plsc.load_gather(eler_t, [sv8 + h])
                    er = plsc.load_gather(eler_t, [dv8 + (h + NH)])
                    e = el + er
                    e = jnp.where(e >= 0.0, e, 0.2 * e)
                    ee = jnp.exp(e)
                    plsc.addupdate_scatter(denp, [dv + h * N_A], ee)
                    plsc.store_scatter(eef, [rr * NH + h], ee)
                return carry2

            lax.fori_loop(0, _CHA // 16, grp, 0)
            return pltpu.async_copy(
                eef, ee_out.at[pl.ds(off * NH, _CHA * NH)], semW)

        def pair(p, carry):
            off = base + p * (2 * _CHA)
            cpA = half(off, sA, dA, eeA)
            cpB = half(off + _CHA, sB, dB, eeB)
            cpA.wait()
            cpB.wait()
            return carry

        lax.fori_loop(0, E_BT // NS // (2 * _CHA), pair, 0)
        pltpu.sync_copy(
            denp,
            dnm.at[pl.ds(pl.multiple_of((c * NS + s) * (NH * N_A), 8),
                         NH * N_A)])

    return k


def _make_sc_gat_b():
    @functools.partial(
        pl.kernel,
        out_type=jax.ShapeDtypeStruct((2 * N_A, HID), f32),
        mesh=plsc.VectorSubcoreMesh(**_MESH),
        compiler_params=pltpu.CompilerParams(needs_layout_passes=False),
        scratch_types=[
            pltpu.VMEM((_CHS, HID), f32),
            pltpu.VMEM((_CHS, HID), f32),
            pltpu.VMEM((_BLK * NH,), f32),
            pltpu.VMEM((_BLK,), i32),
            pltpu.VMEM((_BLK // _CHS, _CHS), i32),
            pltpu.VMEM((_BLK // _CHS, _CHS), i32),
            pltpu.VMEM_SHARED((N_A, HID), f32),
            pltpu.SemaphoreType.DMA,
            pltpu.SemaphoreType.DMA,
        ])
    def k(feats, eehbm, srcs, dsts2, z128, rstu,
          fbufA, fbufB, eeblk, sblk, gblk, dblk,
          acc, semG, semS):
        c = lax.axis_index("c")
        s = lax.axis_index("s")
        r0 = pl.multiple_of(s * 640, 8)
        g0 = pl.multiple_of(c * N_A + s * 640, 8)

        @pl.when(s < NS - 1)
        def _():
            pltpu.sync_copy(z128, acc.at[pl.ds(r0, 640)])

        @pl.when(s == NS - 1)
        def _():
            pltpu.sync_copy(z128.at[pl.ds(0, 400)], acc.at[pl.ds(r0, 400)])

        plsc.subcore_barrier()
        base = c * E_BT + jnp.where(s == 0, 0, 40960 + (s - 1) * 39936)
        nblk = jnp.where(s == 0, 40, 39)
        cbase = c * N_A
        bufs = (fbufA, fbufB)

        def block(b, carry):
            off = base + b * _BLK
            pltpu.sync_copy(srcs.at[pl.ds(off, _BLK)], sblk)
            pltpu.sync_copy(
                dsts2.at[pl.ds(pl.multiple_of(off // _CHS, 8),
                               _BLK // _CHS)], dblk)
            pltpu.sync_copy(eehbm.at[pl.ds(off * NH, _BLK * NH)], eeblk)
            for j in range(_BLK // 16):
                gblk[j // 8, pl.ds((j % 8) * 16, 16)] = (
                    sblk[pl.ds(j * 16, 16)] + cbase)

            def scale(fb, jj):
                def grp(g, carry2):
                    for t in range(16):
                        ei = g * 16 + t
                        eoff = (jj * _CHS + ei) * NH
                        aa = [plsc.load_gather(eeblk, [_full16(eoff + h)])
                              for h in range(NH)]
                        for v in range(HID // 16):
                            fb[ei, pl.ds(v * 16, 16)] = (
                                fb[ei, pl.ds(v * 16, 16)] * aa[v * 16 // HD])
                    return carry2

                lax.fori_loop(0, _CHS // 16, grp, 0)

            cps = [None] * 8
            scs = [None] * 8
            cps[0] = pltpu.async_copy(feats.at[gblk.at[0]], bufs[0], semG)
            for j in range(8):
                if j + 1 < 8:
                    if j >= 1:
                        scs[j - 1].wait()
                    cps[j + 1] = pltpu.async_copy(
                        feats.at[gblk.at[j + 1]], bufs[(j + 1) % 2], semG)
                cps[j].wait()
                scale(bufs[j % 2], j)
                scs[j] = pltpu.async_copy(bufs[j % 2], acc.at[dblk.at[j]],
                                          semS, add=True)
            scs[6].wait()
            scs[7].wait()
            return carry

        lax.fori_loop(0, nblk, block, 0)
        plsc.subcore_barrier()

        @pl.when(s < NS - 1)
        def _():
            pltpu.sync_copy(acc.at[pl.ds(r0, 640)], rstu.at[pl.ds(g0, 640)])

        @pl.when(s == NS - 1)
        def _():
            pltpu.sync_copy(acc.at[pl.ds(r0, 400)], rstu.at[pl.ds(g0, 400)])

    return k


def _make_sc_gin_bt():
    @functools.partial(
        pl.kernel,
        out_type=jax.ShapeDtypeStruct((2 * N_A, HID), f32),
        mesh=plsc.VectorSubcoreMesh(**_MESH),
        compiler_params=pltpu.CompilerParams(needs_layout_passes=False),
        scratch_types=[
            pltpu.VMEM((_CHS, HID), f32),
            pltpu.VMEM((_CHS, HID), f32),
            pltpu.VMEM((_BLK,), i32),
            pltpu.VMEM((_BLK // _CHS, _CHS), i32),
            pltpu.VMEM((_BLK // _CHS, _CHS), i32),
            pltpu.VMEM_SHARED((N_A, HID), f32),
            pltpu.SemaphoreType.DMA,
            pltpu.SemaphoreType.DMA,
        ])
    def k(table, srcs, dsts2, z128, agg,
          fbufA, fbufB, sblk, gblk, dblk, acc, semG, semS):
        c = lax.axis_index("c")
        s = lax.axis_index("s")
        r0 = pl.multiple_of(s * 640, 8)
        g0 = pl.multiple_of(c * N_A + s * 640, 8)

        @pl.when(s < NS - 1)
        def _():
            pltpu.sync_copy(z128, acc.at[pl.ds(r0, 640)])

        @pl.when(s == NS - 1)
        def _():
            pltpu.sync_copy(z128.at[pl.ds(0, 400)], acc.at[pl.ds(r0, 400)])

        plsc.subcore_barrier()
        base = c * E_BT + jnp.where(s == 0, 0, 40960 + (s - 1) * 39936)
        nblk = jnp.where(s == 0, 40, 39)
        cbase = c * N_A
        bufs = (fbufA, fbufB)

        def block(b, carry):
            off = base + b * _BLK
            pltpu.sync_copy(srcs.at[pl.ds(off, _BLK)], sblk)
            pltpu.sync_copy(
                dsts2.at[pl.ds(pl.multiple_of(off // _CHS, 8),
                               _BLK // _CHS)], dblk)
            for j in range(_BLK // 16):
                gblk[j // 8, pl.ds((j % 8) * 16, 16)] = (
                    sblk[pl.ds(j * 16, 16)] + cbase)
            cps = [None] * 8
            scs = [None] * 8
            cps[0] = pltpu.async_copy(table.at[gblk.at[0]], bufs[0], semG)
            for j in range(8):
                if j + 1 < 8:
                    if j >= 1:
                        scs[j - 1].wait()
                    cps[j + 1] = pltpu.async_copy(
                        table.at[gblk.at[j + 1]], bufs[(j + 1) % 2], semG)
                cps[j].wait()
                scs[j] = pltpu.async_copy(bufs[j % 2], acc.at[dblk.at[j]],
                                          semS, add=True)
            scs[6].wait()
            scs[7].wait()
            return carry

        lax.fori_loop(0, nblk, block, 0)
        plsc.subcore_barrier()

        @pl.when(s < NS - 1)
        def _():
            pltpu.sync_copy(acc.at[pl.ds(r0, 640)], agg.at[pl.ds(g0, 640)])

        @pl.when(s == NS - 1)
        def _():
            pltpu.sync_copy(acc.at[pl.ds(r0, 400)], agg.at[pl.ds(g0, 400)])

    return k


def _make_sc_g1max():
    @functools.partial(
        pl.kernel,
        out_type=jax.ShapeDtypeStruct((16 * N_C2 * 16,), f32),
        mesh=plsc.VectorSubcoreMesh(**_MESH),
        compiler_params=pltpu.CompilerParams(needs_layout_passes=False),
        scratch_types=[
            pltpu.VMEM((400, HID), f32),
            pltpu.VMEM((N_C2 * 16,), f32),
            pltpu.VMEM((400,), i32),
        ])
    def k(hA, g1dst, zmax, out, buf, acc, dstb):
        c = lax.axis_index("c")
        s = lax.axis_index("s")
        iota = lax.iota(i32, 16)

        @pl.when(s < 8)
        def _():
            pltpu.sync_copy(zmax, acc)
            cols = s * 16 + iota

            def rchunk(k2, carry):
                pltpu.sync_copy(
                    hA.at[pl.ds(pl.multiple_of(c * N_A + k2 * 400, 8), 400)],
                    buf)
                pltpu.sync_copy(g1dst.at[pl.ds(k2 * 400, 400)], dstb)

                def grp(g, carry2):
                    dv = dstb[pl.ds(g * 16, 16)]
                    nb = g * 16
                    for t in range(16):
                        di = _full16(dv[t]) * 16 + iota
                        v = plsc.load_gather(buf, [_full16(nb + t), cols])
                        cur = plsc.load_gather(acc, [di])
                        plsc.store_scatter(acc, [di], jnp.maximum(cur, v))
                    return carry2

                lax.fori_loop(0, 400 // 16, grp, 0)
                return carry

            lax.fori_loop(0, N_A // 400, rchunk, 0)
            pltpu.sync_copy(
                acc,
                out.at[pl.ds(pl.multiple_of((c * 8 + s) * (N_C2 * 16), 8),
                             N_C2 * 16)])

    return k


N_P = 2048
E_I2P = 32768


def _make_sc_i2(D):
    P = D // HID
    ept = E_I2P // (NC * NS)
    ch = 64

    @functools.partial(
        pl.kernel,
        out_type=jax.ShapeDtypeStruct((2 * P * N_P, HID), f32),
        mesh=plsc.VectorSubcoreMesh(**_MESH),
        compiler_params=pltpu.CompilerParams(needs_layout_passes=False),
        scratch_types=(
            [pltpu.VMEM((ch * P, HID), f32),
             pltpu.VMEM((ch,), i32),
             pltpu.VMEM((ch,), i32)]
            + [pltpu.VMEM((ch,), i32) for _ in range(2 * P)]
            + [pltpu.VMEM_SHARED((P * N_P, HID), f32),
               pltpu.SemaphoreType.DMA]),
    )
    def k(table, srcs, dsts, zrows, out, fbuf, sbuf, dbuf, *rest):
        idxs = rest[:2 * P]
        acc, sem = rest[2 * P], rest[2 * P + 1]
        c = lax.axis_index("c")
        s = lax.axis_index("s")
        rows = P * N_P // NS
        r0 = pl.multiple_of(s * rows, 8)
        g0 = pl.multiple_of(c * (P * N_P) + s * rows, 8)
        pltpu.sync_copy(zrows, acc.at[pl.ds(r0, rows)])
        plsc.subcore_barrier()
        base = (c * NS + s) * ept

        def chunk(kk, carry):
            off = base + kk * ch
            pltpu.sync_copy(srcs.at[pl.ds(off, ch)], sbuf)
            pltpu.sync_copy(dsts.at[pl.ds(off, ch)], dbuf)
            for p in range(P):
                for j in range(ch // 16):
                    sl = pl.ds(j * 16, 16)
                    idxs[p][sl] = sbuf[sl] * P + p
                    idxs[P + p][sl] = dbuf[sl] * P + p
            cps = [pltpu.async_copy(table.at[idxs[p]],
                                    fbuf.at[pl.ds(p * ch, ch)], sem)
                   for p in range(P)]
            for cp in cps:
                cp.wait()
            for p in range(P):
                pltpu.sync_copy(fbuf.at[pl.ds(p * ch, ch)],
                                acc.at[idxs[P + p]], add=True)
            return carry

        lax.fori_loop(0, ept // ch, chunk, 0)
        plsc.subcore_barrier()
        pltpu.sync_copy(acc.at[pl.ds(r0, rows)], out.at[pl.ds(g0, rows)])

    return k


@functools.lru_cache(maxsize=None)
def _sc_kernels():
    return (_make_sc_gat_a(), _make_sc_gat_b(), _make_sc_gin_bt(),
            _make_sc_g1max(), _make_sc_i2(384), _make_sc_i2(128))


def _tc(body, out_shape):
    return pl.pallas_call(body, out_shape=out_shape)


def _tca(x_ref, w_ref, m_ref, feats_ref, eler_ref):
    x = x_ref[...]
    for r in range(2):
        f = jnp.dot(x, w_ref[r], preferred_element_type=f32)
        feats_ref[pl.ds(r * N_A, N_A), :] = f
        eler_ref[pl.ds(r * N_A, N_A), :] = jnp.dot(
            f, m_ref[r], preferred_element_type=f32)


def _tcd(rstu_ref, d1_ref, d2_ref, e4_ref, bias_ref, out_ref):
    for r, dref in ((0, d1_ref), (1, d2_ref)):
        dn = jnp.sum(dref[...], axis=0)
        den = jnp.maximum(
            jax.lax.dot_general(dn, e4_ref[...], (((0,), (0,)), ((), ())),
                                preferred_element_type=f32), 1e-9)
        x = rstu_ref[pl.ds(r * N_A, N_A), :] / den
        out_ref[pl.ds(r * N_A, N_A), :] = jnp.maximum(
            x + bias_ref[pl.ds(r, 1), :], 0.0)


def _bn_mlp(h, w1, b1, g, be, w2, b2):
    h1 = jnp.dot(h, w1, preferred_element_type=f32) + b1
    mu = jnp.mean(h1, axis=0, keepdims=True)
    var = jnp.mean((h1 - mu) ** 2, axis=0, keepdims=True)
    hn = (h1 - mu) / jnp.sqrt(var + 1e-5) * g + be
    h2 = jnp.maximum(hn, 0.0)
    return jnp.dot(h2, w2, preferred_element_type=f32) + b2


def _tcf(hgat_ref, agg_ref, eps_ref, w1_ref, b1_ref, g_ref, be_ref,
         w2_ref, b2_ref, h_ref, m_ref):
    for r in range(2):
        sl = pl.ds(r * N_A, N_A)
        h = ((1.0 + eps_ref[pl.ds(r, 1), :]) * hgat_ref[sl, :]
             + agg_ref[sl, :])
        hr = jnp.maximum(
            _bn_mlp(h, w1_ref[r], b1_ref[pl.ds(r, 1), :], g_ref[pl.ds(r, 1), :],
                    be_ref[pl.ds(r, 1), :], w2_ref[r], b2_ref[pl.ds(r, 1), :]),
            0.0)
        h_ref[sl, :] = hr
        m_ref[pl.ds(r, 1), :] = jnp.mean(hr, axis=0, keepdims=True)


def _tch(hcb_ref, pimg_ref, out_ref):
    for i in range(16):
        off = 128 * (i // 8) + 16 * (i % 8)
        out_ref[pl.ds(0, N_C2), pl.ds(off, 16)] = hcb_ref[
            pl.ds(i * N_C2, N_C2), :]
    out_ref[pl.ds(0, N_C2), pl.ds(256, PI)] = pimg_ref[...]
    out_ref[pl.ds(0, N_C2), pl.ds(256 + PI, 384 - 256 - PI)] = jnp.zeros(
        (N_C2, 384 - 256 - PI), f32)
    out_ref[pl.ds(N_C2, N_P - N_C2), :] = jnp.zeros((N_P - N_C2, 384), f32)


def _tcj(t_ref, agg_ref, eps_ref, w1_ref, b1_ref, g_ref, be_ref,
         w2_ref, b2_ref, out_ref):
    a = agg_ref[pl.ds(0, N_C2), :] + agg_ref[pl.ds(N_P, N_C2), :]
    h = (1.0 + eps_ref[...]) * t_ref[pl.ds(0, N_C2), :] + a
    out_ref[pl.ds(0, N_C2), :] = jnp.maximum(
        _bn_mlp(h, w1_ref[...], b1_ref[...], g_ref[...], be_ref[...],
                w2_ref[...], b2_ref[...]), 0.0)
    out_ref[pl.ds(N_C2, N_P - N_C2), :] = jnp.zeros((N_P - N_C2, HID), f32)


def _tcl(hg1_ref, agg_ref, eps_ref, w1_ref, b1_ref, g_ref, be_ref,
         w2_ref, b2_ref, m_ref, w1o_ref, b1o_ref, w2o_ref, b2o_ref, out_ref):
    a = agg_ref[pl.ds(0, N_C2), :] + agg_ref[pl.ds(N_P, N_C2), :]
    h = (1.0 + eps_ref[...]) * hg1_ref[pl.ds(0, N_C2), :] + a
    hfin = jnp.maximum(
        _bn_mlp(h, w1_ref[...], b1_ref[...], g_ref[...], be_ref[...],
                w2_ref[...], b2_ref[...]), 0.0)
    h2m = jnp.mean(hfin, axis=0, keepdims=True)
    hh = jnp.concatenate(
        [m_ref[pl.ds(0, 1), :], m_ref[pl.ds(1, 1), :], h2m], axis=1)
    o = jnp.maximum(
        jnp.dot(hh, w1o_ref[...], preferred_element_type=f32) + b1o_ref[...],
        0.0)
    out_ref[...] = jnp.dot(o, w2o_ref[...],
                           preferred_element_type=f32) + b2o_ref[...]


def kernel(atom_feats, pimg, params, b1_src, b1_dst, b2_src, b2_dst,
           g1_src, g1_dst, i2_src, i2_dst):
    pb = params["bt"]
    ph = params["h2"]
    po = params["out"]
    (_sc_gat_a, _sc_gat_b, _sc_gin_bt, _sc_g1max, _sc_i2_384,
     _sc_i2_128) = _sc_kernels()

    Wg = jnp.stack([pb[r]["gat"]["W"] for r in range(2)])
    sel = ((jnp.arange(HID)[:, None] // HD)
           == jnp.arange(NH)[None, :]).astype(f32)
    Mlr = jnp.stack([
        jnp.concatenate(
            [sel * pb[r]["gat"]["attn_l"].reshape(-1)[:, None],
             sel * pb[r]["gat"]["attn_r"].reshape(-1)[:, None]], axis=1)
        for r in range(2)])
    e4 = ((jnp.arange(NH)[:, None])
          == (jnp.arange(HID)[None, :] // HD)).astype(f32)
    bias_g = jnp.stack([pb[r]["gat"]["bias"] for r in range(2)])

    def mlp_pack(ps):
        return [jnp.stack([p["mlp"][k] for p in ps]) for k in
                ("W1", "b1", "gamma", "beta", "W2", "b2")]

    gW1, gb1, gg, gbe, gW2, gb2 = mlp_pack([pb[0]["gin"], pb[1]["gin"]])
    geps = jnp.stack([pb[0]["gin"]["eps"], pb[1]["gin"]["eps"]]).reshape(2, 1)

    h0 = ph[0]["mlp"]
    hW1p = jnp.concatenate([h0["W1"], jnp.zeros((384 - 306, HID), f32)], axis=0)
    h1m = ph[1]["mlp"]

    def row(v):
        return v.reshape(1, -1)

    src_flat = jnp.concatenate([b1_src, b2_src]).astype(i32)
    dst_flat = jnp.concatenate([b1_dst, b2_dst]).astype(i32)
    g1d = g1_dst.astype(i32)
    pad = jnp.full((E_I2P - E_I2,), N_P - 1, i32)
    i2s = jnp.concatenate([i2_src.astype(i32), pad])
    i2d = jnp.concatenate([i2_dst.astype(i32), pad])

    z128 = jnp.zeros((640, HID), f32)
    zdn = jnp.zeros((NH * N_A,), f32)
    zmax = jnp.zeros((N_C2 * 16,), f32)
    z384 = jnp.zeros((384, HID), f32)
    z128b = jnp.zeros((128, HID), f32)

    feats, eler = _tc(_tca, (jax.ShapeDtypeStruct((2 * N_A, HID), f32),
                             jax.ShapeDtypeStruct((2 * N_A, 8), f32)))(
        atom_feats, Wg, Mlr)

    eeflat, dnm = _sc_gat_a(eler.reshape(-1), src_flat, dst_flat, zdn)
    dst2 = dst_flat.reshape(-1, _CHS)
    rstu = _sc_gat_b(feats, eeflat, src_flat, dst2, z128)

    dn4 = dnm.reshape(NC, NS, NH, N_A)
    hgat = _tc(_tcd, jax.ShapeDtypeStruct((2 * N_A, HID), f32))(
        rstu, dn4[0], dn4[1], e4, bias_g)

    agg_bt = _sc_gin_bt(hgat, src_flat, dst2, z128)

    hA, means = _tc(_tcf, (jax.ShapeDtypeStruct((2 * N_A, HID), f32),
                           jax.ShapeDtypeStruct((2, HID), f32)))(
        hgat, agg_bt, geps, gW1, gb1, gg, gbe, gW2, gb2)

    hcb = _sc_g1max(hA, g1d, zmax)

    t384 = _tc(_tch, jax.ShapeDtypeStruct((N_P, 384), f32))(
        hcb.reshape(16 * N_C2, 16), pimg)

    agg1 = _sc_i2_384(t384.reshape(3 * N_P, HID), i2s, i2d,
                      z384).reshape(2 * N_P, 384)

    hg1 = _tc(_tcj, jax.ShapeDtypeStruct((N_P, HID), f32))(
        t384, agg1, ph[0]["eps"].reshape(1, 1), hW1p, row(h0["b1"]),
        row(h0["gamma"]), row(h0["beta"]), h0["W2"], row(h0["b2"]))

    agg2 = _sc_i2_128(hg1, i2s, i2d, z128b)

    out = _tc(_tcl, jax.ShapeDtypeStruct((1, HID), f32))(
        hg1, agg2, ph[1]["eps"].reshape(1, 1), h1m["W1"], row(h1m["b1"]),
        row(h1m["gamma"]), row(h1m["beta"]), h1m["W2"], row(h1m["b2"]),
        means, po["W1"], row(po["b1"]), po["W2"], row(po["b2"]))

    return out
